# baseline (speedup 1.0000x reference)
import jax
import jax.numpy as jnp
from jax import lax
from jax.experimental import pallas as pl
from jax.experimental.pallas import tpu as pltpu
from jax.experimental.pallas import tpu_sc as plsc

_SEQ = 200
_D = 64
_L = 16
_NC = 2
_NS = 16
_NW = _NC * _NS
_BB = 128
_SCALE = 8.0


def _body(idx_hbm, table_hbm, pos_hbm, out_hbm,
          idx_blk, idx_t, rows_v, tile_v, pos_v,
          gsem0, gsem1, osem0, osem1):
    gsem = (gsem0, gsem1)
    osem = (osem0, osem1)
    wid = lax.axis_index("s") * _NC + lax.axis_index("c")

    pltpu.sync_copy(idx_hbm.at[pl.ds(wid * _BB, _BB)], idx_blk)
    pltpu.sync_copy(pos_hbm, pos_v)

    lanes = lax.iota(jnp.int32, _L)
    zeros = lanes * 0

    @plsc.parallel_loop(0, _SEQ, step=1, unroll=2)
    def _(s):
        for blk in range(_BB // _L):
            src = plsc.load_gather(idx_blk, [lanes + blk * _L, zeros + s])
            idx_t[s, pl.ds(blk * _L, _L)] = src

    ct_idx, cs_idx = [], []
    for dd in range(_D // _L):
        c = lanes + dd * _L
        ct_idx.append(c // 8)
        cs_idx.append(c % 8)

    def gather_start(slot, s):
        pltpu.async_copy(table_hbm.at[idx_t.at[s]], rows_v.at[slot],
                         gsem[slot])

    def gather_wait(slot):
        pltpu.make_async_copy(table_hbm.at[idx_t.at[0]], rows_v.at[slot],
                              gsem[slot]).wait()

    def out_start(slot, s):
        pltpu.async_copy(tile_v.at[slot], out_hbm.at[s, :, wid], osem[slot])

    def out_wait(slot):
        pltpu.make_async_copy(tile_v.at[slot], out_hbm.at[0, :, wid],
                              osem[slot]).wait()

    gather_start(0, 0)

    def pair(p, carry):
        for b in (0, 1):
            s = 2 * p + b
            nb = 1 - b

            @pl.when(s + 1 < _SEQ)
            def _():
                gather_start(nb, s + 1)

            gather_wait(b)

            @pl.when(s >= 2)
            def _():
                out_wait(b)

            rows = rows_v.at[b]
            tile = tile_v.at[b]
            pos_row = [pos_v[s, pl.ds(dd * _L, _L)] for dd in range(_D // _L)]

            @plsc.parallel_loop(0, _BB, step=1, unroll=4)
            def _(bl):
                bl_splat = zeros + bl
                for dd in range(_D // _L):
                    val = rows[bl, pl.ds(dd * _L, _L)] * _SCALE + pos_row[dd]
                    plsc.store_scatter(tile, [ct_idx[dd], cs_idx[dd], bl_splat],
                                       val)

            out_start(b, s)
        return carry

    lax.fori_loop(0, _SEQ // 2, pair, 0)
    out_wait(0)
    out_wait(1)


def kernel(inputs, token_table, pos_table):
    b, s = inputs.shape
    _, d = token_table.shape
    mesh = plsc.VectorSubcoreMesh(
        core_axis_name="c", subcore_axis_name="s",
        num_cores=_NC, num_subcores=_NS,
    )
    out5 = pl.kernel(
        _body,
        out_type=jax.ShapeDtypeStruct((s, d // 8, b // _BB, 8, _BB),
                                      jnp.float32),
        mesh=mesh,
        compiler_params=pltpu.CompilerParams(use_tc_tiling_on_sc=False,
                                             needs_layout_passes=False),
        scratch_types=[
            pltpu.VMEM((_BB, _SEQ), jnp.int32),
            pltpu.VMEM((_SEQ, _BB), jnp.int32),
            pltpu.VMEM((2, _BB, _D), jnp.float32),
            pltpu.VMEM((2, _D // 8, 8, _BB), jnp.float32),
            pltpu.VMEM((_SEQ, _D), jnp.float32),
        ] + [pltpu.SemaphoreType.DMA] * 4,
    )(inputs, token_table, pos_table)
    return out5.transpose(2, 4, 0, 1, 3).reshape(b, s, d)


# speedup vs baseline: 1.0380x; 1.0380x over previous
import jax
import jax.numpy as jnp
from jax import lax
from jax.experimental import pallas as pl
from jax.experimental.pallas import tpu as pltpu
from jax.experimental.pallas import tpu_sc as plsc

_SEQ = 200
_D = 64
_L = 16
_NC = 2
_NS = 16
_NW = _NC * _NS
_BB = 128
_SCALE = 8.0


def _body(idx_hbm, table_hbm, posb_hbm, out_hbm,
          idx_blk, idx_t, rows_v, tile_v, posb_v,
          gsem0, gsem1, psem0, psem1, osem0, osem1):
    gsem = (gsem0, gsem1)
    psem = (psem0, psem1)
    osem = (osem0, osem1)
    wid = lax.axis_index("s") * _NC + lax.axis_index("c")

    pltpu.sync_copy(idx_hbm.at[pl.ds(wid * _BB, _BB)], idx_blk)

    lanes = lax.iota(jnp.int32, _L)
    zeros = lanes * 0
    row_idx = [lanes + blk * _L for blk in range(_BB // _L)]

    @plsc.parallel_loop(0, _SEQ, step=1, unroll=2)
    def _(s):
        for blk in range(_BB // _L):
            idx_t[s, pl.ds(blk * _L, _L)] = plsc.load_gather(
                idx_blk, [row_idx[blk], zeros + s])

    def gather_start(slot, s):
        pltpu.async_copy(table_hbm.at[idx_t.at[s]], rows_v.at[slot],
                         gsem[slot])

    def gather_wait(slot):
        pltpu.make_async_copy(table_hbm.at[idx_t.at[0]], rows_v.at[slot],
                              gsem[slot]).wait()

    def posb_start(slot, s):
        pltpu.async_copy(posb_hbm.at[s], posb_v.at[slot], psem[slot])

    def posb_wait(slot):
        pltpu.make_async_copy(posb_hbm.at[0], posb_v.at[slot],
                              psem[slot]).wait()

    def out_start(slot, s):
        pltpu.async_copy(tile_v.at[slot], out_hbm.at[s, :, wid], osem[slot])

    def out_wait(slot):
        pltpu.make_async_copy(tile_v.at[slot], out_hbm.at[0, :, wid],
                              osem[slot]).wait()

    gather_start(0, 0)
    posb_start(0, 0)

    def pair(p, carry):
        for b in (0, 1):
            s = 2 * p + b
            nb = 1 - b

            @pl.when(s + 1 < _SEQ)
            def _():
                gather_start(nb, s + 1)
                posb_start(nb, s + 1)

            gather_wait(b)
            posb_wait(b)

            @pl.when(s >= 2)
            def _():
                out_wait(b)

            rows = rows_v.at[b]
            tile = tile_v.at[b]
            posb = posb_v.at[b]

            @plsc.parallel_loop(0, _D, step=1, unroll=2)
            def _(c):
                pb = posb[pl.ds(c * _L, _L)]
                col = zeros + c
                ct = c // 8
                cs = c % 8
                for blk in range(_BB // _L):
                    g = plsc.load_gather(rows, [row_idx[blk], col])
                    tile[ct, cs, pl.ds(blk * _L, _L)] = g * _SCALE + pb

            out_start(b, s)
        return carry

    lax.fori_loop(0, _SEQ // 2, pair, 0)
    out_wait(0)
    out_wait(1)


def kernel(inputs, token_table, pos_table):
    b, s = inputs.shape
    _, d = token_table.shape
    posb = jnp.repeat(pos_table, _L, axis=1)
    mesh = plsc.VectorSubcoreMesh(
        core_axis_name="c", subcore_axis_name="s",
        num_cores=_NC, num_subcores=_NS,
    )
    out5 = pl.kernel(
        _body,
        out_type=jax.ShapeDtypeStruct((s, d // 8, b // _BB, 8, _BB),
                                      jnp.float32),
        mesh=mesh,
        compiler_params=pltpu.CompilerParams(use_tc_tiling_on_sc=False,
                                             needs_layout_passes=False),
        scratch_types=[
            pltpu.VMEM((_BB, _SEQ), jnp.int32),
            pltpu.VMEM((_SEQ, _BB), jnp.int32),
            pltpu.VMEM((2, _BB, _D), jnp.float32),
            pltpu.VMEM((2, _D // 8, 8, _BB), jnp.float32),
            pltpu.VMEM((2, _D * _L), jnp.float32),
        ] + [pltpu.SemaphoreType.DMA] * 6,
    )(inputs, token_table, posb)
    return out5.transpose(2, 4, 0, 1, 3).reshape(b, s, d)
